# baseline (speedup 1.0000x reference)
import functools

import jax
import jax.numpy as jnp
import numpy as np
from jax import lax
from jax.experimental import pallas as pl
from jax.experimental.pallas import tpu as pltpu

B, T, D, M = 16, 128, 64, 8192
N = B * T
N2 = 2 * N
DECAY = 0.99
EPSILON = 1e-05
COMMITMENT_COST = 0.25
MT = 128
NT = M // MT
NCH = 128
NCHUNKS = N // NCH
C2 = 2 * NCHUNKS
BPC = NCH // T
LOG_M = float(np.log(M))
C_EMA = 0.5 * (1.0 - DECAY)


def _dist_tile(x, xsq, e_tile):
    esq = jnp.sum(e_tile * e_tile, axis=1)[None, :]
    g = lax.dot_general(x, e_tile, (((1,), (1,)), ((), ())),
                        preferred_element_type=jnp.float32)
    return (esq + xsq) - 2.0 * g


def _k1a_body(x_ref, e_ref, s_ref, mn_ref, idx_ref):
    t = pl.program_id(1)
    x = x_ref[...]
    xsq = jnp.sum(x * x, axis=1, keepdims=True)
    d = _dist_tile(x, xsq, e_ref[...])
    l = -jnp.sqrt(jnp.maximum(d, 0.0))
    se = jnp.sum(jnp.exp(l), axis=1, keepdims=True)
    tmin = jnp.min(d, axis=1, keepdims=True)
    cols = lax.broadcasted_iota(jnp.int32, (NCH, MT), 1)
    tidx = jnp.min(jnp.where(d == tmin, cols, M), axis=1,
                   keepdims=True) + t * MT

    @pl.when(t == 0)
    def _():
        s_ref[...] = se
        mn_ref[...] = tmin
        idx_ref[...] = tidx

    @pl.when(t != 0)
    def _():
        s_ref[...] = s_ref[...] + se
        mn = mn_ref[...]
        upd = tmin < mn
        mn_ref[...] = jnp.where(upd, tmin, mn)
        idx_ref[...] = jnp.where(upd, tidx, idx_ref[...])


def _run_k1a(x_all, embedding):
    f32 = jnp.float32
    return pl.pallas_call(
        _k1a_body,
        grid=(C2, NT),
        in_specs=[
            pl.BlockSpec((NCH, D), lambda c, t: (c, 0)),
            pl.BlockSpec((MT, D), lambda c, t: (t, 0)),
        ],
        out_specs=[
            pl.BlockSpec((NCH, 1), lambda c, t: (c, 0)),
            pl.BlockSpec((NCH, 1), lambda c, t: (c, 0)),
            pl.BlockSpec((NCH, 1), lambda c, t: (c, 0)),
        ],
        out_shape=(
            jax.ShapeDtypeStruct((N2, 1), f32),
            jax.ShapeDtypeStruct((N2, 1), f32),
            jax.ShapeDtypeStruct((N2, 1), jnp.int32),
        ),
    )(x_all, embedding)


def _k1c_body(x_ref, e_ref, s_ref, xa_ref, xv_ref,
              adj_ref, ph_ref, cons_ref, sc_ref):
    c = pl.program_id(0)
    t = pl.program_id(1)
    x = x_ref[...]
    xsq = jnp.sum(x * x, axis=1, keepdims=True)
    d = _dist_tile(x, xsq, e_ref[...])
    l = -jnp.sqrt(jnp.maximum(d, 0.0))
    p = jnp.exp(l) / s_ref[...]

    ent_cur = jnp.sum(p * jnp.log(p + 1e-5), axis=1, keepdims=True)

    rows = lax.broadcasted_iota(jnp.int32, (BPC, NCH), 0)
    cols = lax.broadcasted_iota(jnp.int32, (BPC, NCH), 1)
    ind = jnp.where(rows == cols // T, 1.0 / T, 0.0).astype(jnp.float32)
    ph = lax.dot_general(ind, p, (((1,), (0,)), ((), ())),
                         preferred_element_type=jnp.float32)
    ph_tok = lax.dot_general(jnp.where(rows == cols // T, 1.0, 0.0), ph,
                             (((0,), (0,)), ((), ())),
                             preferred_element_type=jnp.float32)
    cons_cur = jnp.sum(jnp.abs(p - ph_tok), axis=1, keepdims=True)

    ph_ref[...] = jnp.concatenate(
        [ph, jnp.zeros((8 - BPC, MT), jnp.float32)], axis=0)

    zero = jnp.zeros((NCH, 1), jnp.float32)
    prev_c = jnp.where(t == 0, zero, cons_ref[...])
    cons_ref[...] = prev_c + cons_cur

    prev_e = jnp.where(t == 0, zero, adj_ref[...])
    acc = prev_e + ent_cur

    @pl.when(t != NT - 1)
    def _():
        adj_ref[...] = acc

    @pl.when(t == NT - 1)
    def _():
        adj = 1.0 + acc / LOG_M
        adj_ref[...] = adj
        scale = jnp.where(c < NCHUNKS, C_EMA, DECAY * C_EMA)
        sc_ref[...] = (scale * adj) * (xa_ref[...] + xv_ref[...])


def _run_k1c(x_all, embedding, s_all, a_flat, v_flat):
    f32 = jnp.float32
    return pl.pallas_call(
        _k1c_body,
        grid=(C2, NT),
        in_specs=[
            pl.BlockSpec((NCH, D), lambda c, t: (c, 0)),
            pl.BlockSpec((MT, D), lambda c, t: (t, 0)),
            pl.BlockSpec((NCH, 1), lambda c, t: (c, 0)),
            pl.BlockSpec((NCH, D), lambda c, t: (lax.rem(c, NCHUNKS), 0)),
            pl.BlockSpec((NCH, D), lambda c, t: (lax.rem(c, NCHUNKS), 0)),
        ],
        out_specs=[
            pl.BlockSpec((NCH, 1), lambda c, t: (c, 0)),
            pl.BlockSpec((8, MT), lambda c, t: (c, t)),
            pl.BlockSpec((NCH, 1), lambda c, t: (c, 0)),
            pl.BlockSpec((NCH, D), lambda c, t: (c, 0)),
        ],
        out_shape=(
            jax.ShapeDtypeStruct((N2, 1), f32),
            jax.ShapeDtypeStruct((C2 * 8, M), f32),
            jax.ShapeDtypeStruct((N2, 1), f32),
            jax.ShapeDtypeStruct((N2, D), f32),
        ),
    )(x_all, embedding, s_all, a_flat, v_flat)


def _k1e_body(mn_ref, cons_ref, ph_a_ref, ph_v_ref,
              loss_a_ref, loss_v_ref, cons_a_ref, cons_v_ref, cmcm_ref):
    lscale = COMMITMENT_COST * 2.0 / (N * D)
    loss_a_ref[...] = lscale * jnp.sum(mn_ref[pl.ds(0, N), :],
                                       keepdims=True).reshape(1, 1)
    loss_v_ref[...] = lscale * jnp.sum(mn_ref[pl.ds(N, N), :],
                                       keepdims=True).reshape(1, 1)
    cons_a_ref[...] = (1.0 / B) * jnp.sum(cons_ref[pl.ds(0, N), :],
                                          keepdims=True).reshape(1, 1)
    cons_v_ref[...] = (1.0 / B) * jnp.sum(cons_ref[pl.ds(N, N), :],
                                          keepdims=True).reshape(1, 1)

    ph_a = ph_a_ref[...]
    ph_v = ph_v_ref[...]
    t1 = lax.dot_general(ph_a, jnp.log(ph_v + 1e-10), (((1,), (1,)), ((), ())),
                         preferred_element_type=jnp.float32)
    t2 = lax.dot_general(ph_v, jnp.log(ph_a + 1e-10), (((1,), (1,)), ((), ())),
                         preferred_element_type=jnp.float32)
    scode = t1 + t2
    maxs = jnp.max(-scode)
    es = jnp.exp(scode + maxs)
    rowsum = jnp.sum(es, axis=1, keepdims=True)
    r = lax.broadcasted_iota(jnp.int32, (B, B), 0)
    cc = lax.broadcasted_iota(jnp.int32, (B, B), 1)
    diag = jnp.sum(jnp.where(r == cc, es, 0.0), axis=1, keepdims=True)
    lcmcm = -jnp.sum(jnp.log(diag / (rowsum + EPSILON))) / B
    cmcm_ref[...] = jnp.full((1, 1), 0.5, jnp.float32) * lcmcm


def _run_k1e(mn_all, cons_raw, ph_a, ph_v):
    f32 = jnp.float32
    scalar = jax.ShapeDtypeStruct((1, 1), f32)
    return pl.pallas_call(
        _k1e_body,
        out_shape=(scalar, scalar, scalar, scalar, scalar),
    )(mn_all, cons_raw, ph_a, ph_v)


def kernel(audio_semantic, video_semantic, epoch, embedding, ema_count,
           ema_weight, coefficients):
    a_flat = audio_semantic.reshape(N, D)
    v_flat = video_semantic.reshape(N, D)
    x_all = jnp.concatenate([a_flat, v_flat], axis=0)

    s_all, mn_all, idx_all = _run_k1a(x_all, embedding)
    adj_all, ph_pad, cons_raw, sc_all = _run_k1c(
        x_all, embedding, s_all, a_flat, v_flat)

    ph_all = ph_pad.reshape(C2, 8, M)[:, :BPC, :].reshape(2 * B, M)
    a_loss, v_loss, a_cons, v_cons, cmcm_raw = _run_k1e(
        mn_all, cons_raw, ph_all[:B], ph_all[B:])

    idx_a = idx_all[:N].reshape(N)
    idx_v = idx_all[N:].reshape(N)
    adj_a1 = adj_all[:N].reshape(N)
    adj_v1 = adj_all[N:].reshape(N)
    scaled_a = sc_all[:N]
    scaled_v = sc_all[N:]

    a_quant = jnp.take(embedding, idx_a, axis=0)
    v_quant = jnp.take(embedding, idx_v, axis=0)

    sv = jnp.zeros((M,), jnp.float32).at[idx_v].add(adj_v1)
    sa = jnp.zeros((M,), jnp.float32).at[idx_a].add(adj_a1)
    acc = jnp.zeros((M, D), jnp.float32).at[idx_v].add(scaled_v)
    acc = acc.at[idx_a].add(scaled_a)

    ema_count1 = DECAY * ema_count + (1.0 - DECAY) * sv
    n1 = jnp.sum(ema_count1)
    ema_count1 = (ema_count1 + EPSILON) / (n1 + M * EPSILON) * n1
    ema_count2 = DECAY * ema_count1 + (1.0 - DECAY) * sa
    n2 = jnp.sum(ema_count2)
    ema_count2 = (ema_count2 + EPSILON) / (n2 + M * EPSILON) * n2
    ema_weight2 = (DECAY * DECAY) * ema_weight + acc
    embedding2 = ema_weight2 / ema_count2[:, None]

    mode_fn = jax.vmap(lambda rr: jnp.argmax(jnp.bincount(rr, length=M)))
    equal_num = jnp.sum(mode_fn(idx_a.reshape(B, T)) ==
                        mode_fn(idx_v.reshape(B, T)))

    cmcm_loss = jnp.where(epoch < 10, 0.0, cmcm_raw.reshape(()))

    a_q = audio_semantic + (a_quant.reshape(audio_semantic.shape)
                            - audio_semantic)
    v_q = video_semantic + (v_quant.reshape(video_semantic.shape)
                            - video_semantic)
    return (a_q, v_q,
            a_loss.reshape(()), v_loss.reshape(()), cmcm_loss,
            a_cons.reshape(()), v_cons.reshape(()),
            equal_num, embedding2)
